# baseline (speedup 1.0000x reference)
import jax
import jax.numpy as jnp
from jax.experimental import pallas as pl
from jax.experimental.pallas import tpu as pltpu

B = 32
N = 128
H = 64
NL = 4


def _silu(x):
    return x * jax.nn.sigmoid(x)


def _egnn_kernel(inp_ref, emb_w_ref, emb_b_ref,
                 ws_ref, wr_ref, wrad_ref, eb1_ref, ew2_ref, eb2_ref,
                 nw1h_ref, nw1a_ref, nb1_ref, nw2_ref, nb2_ref,
                 cw1_ref, cb1_ref, cw2_ref,
                 vw1_ref, vb1_ref, vw2_ref, vb2_ref,
                 out_ref):
    inp = inp_ref[0]
    x = inp[:, 0:3]
    vel = inp[:, 3:6]

    hv = jnp.sqrt(jnp.sum(vel * vel, axis=-1, keepdims=True))
    h = hv * emb_w_ref[:] + emb_b_ref[:]

    inv_count = 1.0 / float(N - 1)
    v = vel
    for l in range(NL):
        ws = ws_ref[l]
        wr = wr_ref[l]
        wrad = wrad_ref[l]
        eb1 = eb1_ref[l]
        ew2 = ew2_ref[l]
        eb2 = eb2_ref[l]

        hs = jnp.dot(h, ws, preferred_element_type=jnp.float32)
        hr = jnp.dot(h, wr, preferred_element_type=jnp.float32) + eb1

        td = _silu(hs + hr)
        ed = _silu(jnp.dot(td, ew2, preferred_element_type=jnp.float32) + eb2)

        xT = jnp.transpose(x)
        cds = [x[:, k:k + 1] - xT[k:k + 1, :] for k in range(3)]
        radial2 = cds[0] * cds[0] + cds[1] * cds[1] + cds[2] * cds[2]

        rad_b2 = radial2.astype(jnp.bfloat16).astype(jnp.float32)
        rad3 = jnp.transpose(rad_b2.reshape(N, 1, N), (0, 2, 1))
        wrad_b = wrad.astype(jnp.bfloat16).astype(jnp.float32)
        t3 = _silu(hs[:, None, :] + hr[None, :, :]
                   + rad3 * wrad_b[None, :, :])
        tf = t3.reshape(N * N, H)
        ef = _silu(jnp.dot(tf, ew2, preferred_element_type=jnp.float32) + eb2)
        e3 = ef.reshape(N, N, H)

        hagg = jnp.sum(e3, axis=0) - ed

        c1 = _silu(jnp.dot(ef, cw1_ref[l],
                           preferred_element_type=jnp.float32) + cb1_ref[l])
        cmf = jnp.dot(c1, cw2_ref[l], preferred_element_type=jnp.float32)
        cm2 = jnp.transpose(cmf.reshape(N, N, 1), (0, 2, 1)).reshape(N, N)
        aggT = jnp.concatenate(
            [jnp.sum(jnp.clip(cds[k] * cm2, -100.0, 100.0),
                     axis=0, keepdims=True) for k in range(3)], axis=0)
        agg = jnp.transpose(aggT) * inv_count

        vmul = (jnp.dot(_silu(jnp.dot(h, vw1_ref[l],
                                      preferred_element_type=jnp.float32)
                              + vb1_ref[l]),
                        vw2_ref[l], preferred_element_type=jnp.float32)
                + vb2_ref[l])
        new_vel = agg + vmul * vel
        x = x + new_vel
        v = new_vel

        hmid = _silu(jnp.dot(h, nw1h_ref[l], preferred_element_type=jnp.float32)
                     + jnp.dot(hagg, nw1a_ref[l], preferred_element_type=jnp.float32)
                     + nb1_ref[l])
        h = h + jnp.dot(hmid, nw2_ref[l],
                        preferred_element_type=jnp.float32) + nb2_ref[l]

    out_ref[0] = jnp.concatenate([x, v], axis=-1)


def kernel(inputs, params, send_edges, recv_edges):
    del send_edges, recv_edges
    layers = params["layers"]

    def stack(f):
        return jnp.stack([f(lp) for lp in layers])

    ws = stack(lambda lp: lp["ew1"][0:H, :])
    wr = stack(lambda lp: lp["ew1"][H:2 * H, :])
    wrad = stack(lambda lp: lp["ew1"][2 * H:2 * H + 1, :])
    eb1 = stack(lambda lp: lp["eb1"][None, :])
    ew2 = stack(lambda lp: lp["ew2"])
    eb2 = stack(lambda lp: lp["eb2"][None, :])
    nw1h = stack(lambda lp: lp["nw1"][0:H, :])
    nw1a = stack(lambda lp: lp["nw1"][H:2 * H, :])
    nb1 = stack(lambda lp: lp["nb1"][None, :])
    nw2 = stack(lambda lp: lp["nw2"])
    nb2 = stack(lambda lp: lp["nb2"][None, :])
    cw1 = stack(lambda lp: lp["cw1"])
    cb1 = stack(lambda lp: lp["cb1"][None, :])
    cw2 = stack(lambda lp: lp["cw2"])
    vw1 = stack(lambda lp: lp["vw1"])
    vb1 = stack(lambda lp: lp["vb1"][None, :])
    vw2 = stack(lambda lp: lp["vw2"])
    vb2 = stack(lambda lp: lp["vb2"][None, :])

    emb_w = params["emb_w"]
    emb_b = params["emb_b"][None, :]

    def w_spec(a):
        nd = a.ndim
        return pl.BlockSpec(a.shape, lambda b, _n=nd: (0,) * _n)

    weights = (emb_w, emb_b, ws, wr, wrad, eb1, ew2, eb2,
               nw1h, nw1a, nb1, nw2, nb2, cw1, cb1, cw2,
               vw1, vb1, vw2, vb2)

    return pl.pallas_call(
        _egnn_kernel,
        grid=(B,),
        in_specs=[pl.BlockSpec((1, N, 6), lambda b: (b, 0, 0))]
                 + [w_spec(a) for a in weights],
        out_specs=pl.BlockSpec((1, N, 6), lambda b: (b, 0, 0)),
        out_shape=jax.ShapeDtypeStruct((B, N, 6), jnp.float32),
        compiler_params=pltpu.CompilerParams(
            dimension_semantics=("parallel",)),
    )(inputs, *weights)


# speedup vs baseline: 1.6862x; 1.6862x over previous
import jax
import jax.numpy as jnp
from jax.experimental import pallas as pl
from jax.experimental.pallas import tpu as pltpu

B = 32
N = 128
H = 64
NL = 4


def _silu(x):
    xh = 0.5 * x
    return xh * jnp.tanh(xh) + xh


def _egnn_kernel(inp_ref, emb_w_ref, emb_b_ref,
                 ws_ref, wr_ref, wrad_ref, eb1_ref, ew2_ref, eb2_ref,
                 nw1h_ref, nw1a_ref, nb1_ref, nw2_ref, nb2_ref,
                 cw1_ref, cb1_ref, cw2_ref,
                 vw1_ref, vb1_ref, vw2_ref, vb2_ref,
                 out_ref):
    inp = inp_ref[0]
    x = inp[:, 0:3]
    vel = inp[:, 3:6]

    hv = jnp.sqrt(jnp.sum(vel * vel, axis=-1, keepdims=True))
    h = hv * emb_w_ref[:] + emb_b_ref[:]

    inv_count = 1.0 / float(N - 1)
    v = vel
    for l in range(NL):
        ws = ws_ref[l]
        wr = wr_ref[l]
        wrad = wrad_ref[l]
        eb1 = eb1_ref[l]
        ew2 = ew2_ref[l]
        eb2 = eb2_ref[l]

        hs = jnp.dot(h, ws, preferred_element_type=jnp.float32)
        hr = jnp.dot(h, wr, preferred_element_type=jnp.float32) + eb1

        td = _silu(hs + hr)
        ed = _silu(jnp.dot(td, ew2, preferred_element_type=jnp.float32) + eb2)

        cd3 = x[:, None, :] - x[None, :, :]
        radial = jnp.sum(cd3 * cd3, axis=-1, keepdims=True)

        rad_b = radial.astype(jnp.bfloat16).astype(jnp.float32)
        wrad_b = wrad.astype(jnp.bfloat16).astype(jnp.float32)
        t3 = _silu(hs[:, None, :] + hr[None, :, :]
                   + rad_b * wrad_b[None, :, :])
        tf = t3.reshape(N * N, H)
        ef = _silu(jnp.dot(tf, ew2, preferred_element_type=jnp.float32) + eb2)
        e3 = ef.reshape(N, N, H)

        hagg = jnp.sum(e3, axis=0) - ed

        c1 = _silu(jnp.dot(ef, cw1_ref[l],
                           preferred_element_type=jnp.float32) + cb1_ref[l])
        cmf = jnp.dot(c1, cw2_ref[l], preferred_element_type=jnp.float32)
        cm3 = cmf.reshape(N, N, 1)
        trans3 = jnp.clip(cd3 * cm3, -100.0, 100.0)
        agg = jnp.sum(trans3, axis=0) * inv_count

        vmul = (jnp.dot(_silu(jnp.dot(h, vw1_ref[l],
                                      preferred_element_type=jnp.float32)
                              + vb1_ref[l]),
                        vw2_ref[l], preferred_element_type=jnp.float32)
                + vb2_ref[l])
        new_vel = agg + vmul * vel
        x = x + new_vel
        v = new_vel

        hmid = _silu(jnp.dot(h, nw1h_ref[l], preferred_element_type=jnp.float32)
                     + jnp.dot(hagg, nw1a_ref[l], preferred_element_type=jnp.float32)
                     + nb1_ref[l])
        h = h + jnp.dot(hmid, nw2_ref[l],
                        preferred_element_type=jnp.float32) + nb2_ref[l]

    out_ref[0] = jnp.concatenate([x, v], axis=-1)


def kernel(inputs, params, send_edges, recv_edges):
    del send_edges, recv_edges
    layers = params["layers"]

    def stack(f):
        return jnp.stack([f(lp) for lp in layers])

    ws = stack(lambda lp: lp["ew1"][0:H, :])
    wr = stack(lambda lp: lp["ew1"][H:2 * H, :])
    wrad = stack(lambda lp: lp["ew1"][2 * H:2 * H + 1, :])
    eb1 = stack(lambda lp: lp["eb1"][None, :])
    ew2 = stack(lambda lp: lp["ew2"])
    eb2 = stack(lambda lp: lp["eb2"][None, :])
    nw1h = stack(lambda lp: lp["nw1"][0:H, :])
    nw1a = stack(lambda lp: lp["nw1"][H:2 * H, :])
    nb1 = stack(lambda lp: lp["nb1"][None, :])
    nw2 = stack(lambda lp: lp["nw2"])
    nb2 = stack(lambda lp: lp["nb2"][None, :])
    cw1 = stack(lambda lp: lp["cw1"])
    cb1 = stack(lambda lp: lp["cb1"][None, :])
    cw2 = stack(lambda lp: lp["cw2"])
    vw1 = stack(lambda lp: lp["vw1"])
    vb1 = stack(lambda lp: lp["vb1"][None, :])
    vw2 = stack(lambda lp: lp["vw2"])
    vb2 = stack(lambda lp: lp["vb2"][None, :])

    emb_w = params["emb_w"]
    emb_b = params["emb_b"][None, :]

    def w_spec(a):
        nd = a.ndim
        return pl.BlockSpec(a.shape, lambda b, _n=nd: (0,) * _n)

    weights = (emb_w, emb_b, ws, wr, wrad, eb1, ew2, eb2,
               nw1h, nw1a, nb1, nw2, nb2, cw1, cb1, cw2,
               vw1, vb1, vw2, vb2)

    return pl.pallas_call(
        _egnn_kernel,
        grid=(B,),
        in_specs=[pl.BlockSpec((1, N, 6), lambda b: (b, 0, 0))]
                 + [w_spec(a) for a in weights],
        out_specs=pl.BlockSpec((1, N, 6), lambda b: (b, 0, 0)),
        out_shape=jax.ShapeDtypeStruct((B, N, 6), jnp.float32),
        compiler_params=pltpu.CompilerParams(
            dimension_semantics=("parallel",)),
    )(inputs, *weights)


# speedup vs baseline: 1.8507x; 1.0976x over previous
import jax
import jax.numpy as jnp
from jax.experimental import pallas as pl
from jax.experimental.pallas import tpu as pltpu

B = 32
N = 128
H = 64
NL = 4
M = N // 2


def _silu(x):
    xh = 0.5 * x
    return xh * jnp.tanh(xh) + xh


def _egnn_kernel(inp_ref, emb_w_ref, emb_b_ref,
                 ws_ref, wr_ref, wradl_ref, wradh_ref, eb1_ref,
                 ew2_ref, eb2_ref, ew2p_ref, eb2p_ref,
                 nw1h_ref, nw1a_ref, nb1_ref, nw2_ref, nb2_ref,
                 cw1p_ref, cb1p_ref, cw2p_ref,
                 vw1_ref, vb1_ref, vw2_ref, vb2_ref,
                 out_ref):
    inp = inp_ref[0]
    x = inp[:, 0:3]
    vel = inp[:, 3:6]

    hv = jnp.sqrt(jnp.sum(vel * vel, axis=-1, keepdims=True))
    h = hv * emb_w_ref[:] + emb_b_ref[:]

    def dot(a, b):
        return jnp.dot(a, b, preferred_element_type=jnp.float32)

    inv_count = 1.0 / float(N - 1)
    v = vel
    for l in range(NL):
        hs = dot(h, ws_ref[l])
        hr = dot(h, wr_ref[l]) + eb1_ref[l]
        hs2 = jnp.concatenate([hs, hs], axis=1)
        hr_p = jnp.concatenate([hr[0:M, :], hr[M:N, :]], axis=1)

        td = _silu(hs + hr)
        ed = _silu(dot(td, ew2_ref[l]) + eb2_ref[l])

        cd3 = x[:, None, :] - x[None, :, :]
        radial = jnp.sum(cd3 * cd3, axis=-1, keepdims=True)

        rad_b = radial.astype(jnp.bfloat16).astype(jnp.float32)
        t3 = _silu(hs2[:, None, :] + hr_p[None, :, :]
                   + rad_b[:, 0:M, :] * wradl_ref[l][None, :, :]
                   + rad_b[:, M:N, :] * wradh_ref[l][None, :, :])
        tf = t3.reshape(N * M, 2 * H)
        ef = _silu(dot(tf, ew2p_ref[l]) + eb2p_ref[l])

        hagg_p = jnp.sum(ef.reshape(N, M, 2 * H), axis=0)
        hagg = jnp.concatenate([hagg_p[:, 0:H], hagg_p[:, H:2 * H]],
                               axis=0) - ed

        c1 = _silu(dot(ef, cw1p_ref[l]) + cb1p_ref[l])
        cmp = dot(c1, cw2p_ref[l])
        cm3 = jnp.concatenate([cmp[:, 0:1].reshape(N, M, 1),
                               cmp[:, 1:2].reshape(N, M, 1)], axis=1)
        trans3 = jnp.clip(cd3 * cm3, -100.0, 100.0)
        agg = jnp.sum(trans3, axis=0) * inv_count

        vmul = dot(_silu(dot(h, vw1_ref[l]) + vb1_ref[l]),
                   vw2_ref[l]) + vb2_ref[l]
        new_vel = agg + vmul * vel
        x = x + new_vel
        v = new_vel

        hmid = _silu(dot(h, nw1h_ref[l]) + dot(hagg, nw1a_ref[l])
                     + nb1_ref[l])
        h = h + dot(hmid, nw2_ref[l]) + nb2_ref[l]

    out_ref[0] = jnp.concatenate([x, v], axis=-1)


def kernel(inputs, params, send_edges, recv_edges):
    del send_edges, recv_edges
    layers = params["layers"]

    def stack(f):
        return jnp.stack([f(lp) for lp in layers])

    def blockdiag(w):
        z = jnp.zeros_like(w)
        return jnp.concatenate(
            [jnp.concatenate([w, z], axis=1),
             jnp.concatenate([z, w], axis=1)], axis=0)

    ws = stack(lambda lp: lp["ew1"][0:H, :])
    wr = stack(lambda lp: lp["ew1"][H:2 * H, :])

    def wrad_b(lp):
        w = lp["ew1"][2 * H:2 * H + 1, :]
        return w.astype(jnp.bfloat16).astype(jnp.float32)

    zh = jnp.zeros((1, H), jnp.float32)
    wradl = stack(lambda lp: jnp.concatenate([wrad_b(lp), zh], axis=1))
    wradh = stack(lambda lp: jnp.concatenate([zh, wrad_b(lp)], axis=1))
    eb1 = stack(lambda lp: lp["eb1"][None, :])
    ew2 = stack(lambda lp: lp["ew2"])
    eb2 = stack(lambda lp: lp["eb2"][None, :])
    ew2p = stack(lambda lp: blockdiag(lp["ew2"]))
    eb2p = stack(lambda lp: jnp.tile(lp["eb2"][None, :], (1, 2)))
    nw1h = stack(lambda lp: lp["nw1"][0:H, :])
    nw1a = stack(lambda lp: lp["nw1"][H:2 * H, :])
    nb1 = stack(lambda lp: lp["nb1"][None, :])
    nw2 = stack(lambda lp: lp["nw2"])
    nb2 = stack(lambda lp: lp["nb2"][None, :])
    cw1p = stack(lambda lp: blockdiag(lp["cw1"]))
    cb1p = stack(lambda lp: jnp.tile(lp["cb1"][None, :], (1, 2)))
    cw2p = stack(lambda lp: blockdiag(lp["cw2"]))
    vw1 = stack(lambda lp: lp["vw1"])
    vb1 = stack(lambda lp: lp["vb1"][None, :])
    vw2 = stack(lambda lp: lp["vw2"])
    vb2 = stack(lambda lp: lp["vb2"][None, :])

    emb_w = params["emb_w"]
    emb_b = params["emb_b"][None, :]

    def w_spec(a):
        nd = a.ndim
        return pl.BlockSpec(a.shape, lambda b, _n=nd: (0,) * _n)

    weights = (emb_w, emb_b,
               ws, wr, wradl, wradh, eb1, ew2, eb2, ew2p, eb2p,
               nw1h, nw1a, nb1, nw2, nb2, cw1p, cb1p, cw2p,
               vw1, vb1, vw2, vb2)

    return pl.pallas_call(
        _egnn_kernel,
        grid=(B,),
        in_specs=[pl.BlockSpec((1, N, 6), lambda b: (b, 0, 0))]
                 + [w_spec(a) for a in weights],
        out_specs=pl.BlockSpec((1, N, 6), lambda b: (b, 0, 0)),
        out_shape=jax.ShapeDtypeStruct((B, N, 6), jnp.float32),
        compiler_params=pltpu.CompilerParams(
            dimension_semantics=("parallel",)),
    )(inputs, *weights)
